# baseline (speedup 1.0000x reference)
import functools

import jax
import jax.numpy as jnp
from jax import lax
from jax.experimental import pallas as pl
from jax.experimental.pallas import tpu as pltpu
from jax.experimental.pallas import tpu_sc as plsc

_B = 16384
_D = 128
_H = 1024
_NC = 2
_NS = 16
_NW = _NC * _NS
_BPW = _B // _NW
_CH = 128
_NCH = _BPW // _CH

_BLK = 2048


_NBUF = 3


def _gather_body(uidx_hbm, vidx_hbm, user_hbm, movie_hbm, h_out,
                 idx_v, bufs, sem_g, sem_o):
    wid = lax.axis_index("s") * _NC + lax.axis_index("c")
    base = wid * _BPW
    row0 = wid * _NCH
    pltpu.sync_copy(uidx_hbm.at[pl.ds(row0, _NCH)],
                    idx_v.at[pl.ds(0, _NCH)])
    pltpu.sync_copy(vidx_hbm.at[pl.ds(row0, _NCH)],
                    idx_v.at[pl.ds(_NCH, _NCH)])
    out_handles = {}
    for t in range(2 * _NCH):
        tbl = user_hbm if t < _NCH else movie_hbm
        col = 0 if t < _NCH else _D
        chunk = t % _NCH
        b = t % _NBUF
        if t >= _NBUF:
            out_handles[t - _NBUF].wait()
        pltpu.async_copy(tbl.at[idx_v.at[t]], bufs.at[b], sem_g).wait()
        out_handles[t] = pltpu.async_copy(
            bufs.at[b],
            h_out.at[pl.ds(base + chunk * _CH, _CH), pl.ds(col, _D)],
            sem_o)
    for t in range(2 * _NCH - _NBUF, 2 * _NCH):
        out_handles[t].wait()


@jax.jit
def _gather(uidx, vidx, user_emb, movie_emb):
    mesh = plsc.VectorSubcoreMesh(core_axis_name="c", subcore_axis_name="s",
                                  num_cores=_NC, num_subcores=_NS)
    return pl.kernel(
        _gather_body,
        out_type=jax.ShapeDtypeStruct((_B, 2 * _D), jnp.float32),
        mesh=mesh,
        scratch_types=[
            pltpu.VMEM((2 * _NCH, _CH), jnp.int32),
            pltpu.VMEM((_NBUF, _CH, _D), jnp.float32),
            pltpu.SemaphoreType.DMA,
            pltpu.SemaphoreType.DMA,
        ],
    )(uidx, vidx, user_emb, movie_emb)


def _mlp_body(h_ref, w1_ref, b1_ref, w2_ref, b2_ref, out_ref):
    hin = jnp.maximum(h_ref[...].astype(jnp.bfloat16), 0)
    h = jnp.dot(hin, w1_ref[...], preferred_element_type=jnp.float32)
    h = jnp.maximum(h + b1_ref[...], 0.0).astype(jnp.bfloat16)
    logit = jax.lax.dot_general(w2_ref[...], h,
                                (((1,), (1,)), ((), ())),
                                preferred_element_type=jnp.float32)
    logit = logit + b2_ref[0, 0]
    out_ref[...] = 5.5 * jax.nn.sigmoid(-logit)


@jax.jit
def _mlp(H, w1, b1, w2, b2):
    grid = (_B // _BLK,)
    return pl.pallas_call(
        _mlp_body,
        grid=grid,
        in_specs=[
            pl.BlockSpec((_BLK, 2 * _D), lambda i: (i, 0)),
            pl.BlockSpec((2 * _D, _H), lambda i: (0, 0)),
            pl.BlockSpec((1, _H), lambda i: (0, 0)),
            pl.BlockSpec((1, _H), lambda i: (0, 0)),
            pl.BlockSpec((1, 1), lambda i: (0, 0)),
        ],
        out_specs=pl.BlockSpec((1, _BLK), lambda i: (0, i)),
        out_shape=jax.ShapeDtypeStruct((1, _B), jnp.float32),
    )(H, w1, b1, w2, b2)


def kernel(x, user_emb, movie_emb, W1, b1, W2, b2):
    uidx = x[:, 0].reshape(_B // _CH, _CH)
    vidx = x[:, 1].reshape(_B // _CH, _CH)
    H = _gather(uidx, vidx, user_emb, movie_emb)
    out = _mlp(H, W1.astype(jnp.bfloat16), b1.reshape(1, _H),
               W2.reshape(1, _H).astype(jnp.bfloat16), b2.reshape(1, 1))
    return out.reshape(_B)


# speedup vs baseline: 1.0536x; 1.0536x over previous
import functools

import jax
import jax.numpy as jnp
from jax import lax
from jax.experimental import pallas as pl
from jax.experimental.pallas import tpu as pltpu
from jax.experimental.pallas import tpu_sc as plsc

_B = 16384
_D = 128
_H = 1024
_NC = 2
_NS = 16
_NW = _NC * _NS
_BPW = _B // _NW
_CH = 128
_NCH = _BPW // _CH

_BLK = 2048


def _gather_body(uidx_hbm, vidx_hbm, user_hbm, movie_hbm, h_out,
                 idx_v, bufs, sem_g, sem_o):
    wid = lax.axis_index("s") * _NC + lax.axis_index("c")
    base = wid * _BPW
    row0 = wid * _NCH
    pltpu.sync_copy(uidx_hbm.at[pl.ds(row0, _NCH)],
                    idx_v.at[pl.ds(0, _NCH)])
    pltpu.sync_copy(vidx_hbm.at[pl.ds(row0, _NCH)],
                    idx_v.at[pl.ds(_NCH, _NCH)])
    def _out(t):
        col = 0 if t < _NCH else _D
        chunk = t % _NCH
        return pltpu.async_copy(
            bufs.at[chunk],
            h_out.at[pl.ds(base + chunk * _CH, _CH), pl.ds(col, _D)],
            sem_o)

    g_u = [pltpu.async_copy(user_hbm.at[idx_v.at[c]], bufs.at[c], sem_g)
           for c in range(_NCH)]
    o_u = []
    for c in range(_NCH):
        g_u[c].wait()
        o_u.append(_out(c))
    g_m = []
    for c in range(_NCH):
        o_u[c].wait()
        g_m.append(pltpu.async_copy(movie_hbm.at[idx_v.at[_NCH + c]],
                                    bufs.at[c], sem_g))
    o_m = []
    for c in range(_NCH):
        g_m[c].wait()
        o_m.append(_out(_NCH + c))
    for c in range(_NCH):
        o_m[c].wait()


@jax.jit
def _gather(uidx, vidx, user_emb, movie_emb):
    mesh = plsc.VectorSubcoreMesh(core_axis_name="c", subcore_axis_name="s",
                                  num_cores=_NC, num_subcores=_NS)
    return pl.kernel(
        _gather_body,
        out_type=jax.ShapeDtypeStruct((_B, 2 * _D), jnp.float32),
        mesh=mesh,
        scratch_types=[
            pltpu.VMEM((2 * _NCH, _CH), jnp.int32),
            pltpu.VMEM((_NCH, _CH, _D), jnp.float32),
            pltpu.SemaphoreType.DMA,
            pltpu.SemaphoreType.DMA,
        ],
    )(uidx, vidx, user_emb, movie_emb)


def _mlp_body(h_ref, w1_ref, b1_ref, w2_ref, b2_ref, out_ref):
    hin = jnp.maximum(h_ref[...].astype(jnp.bfloat16), 0)
    h = jnp.dot(hin, w1_ref[...], preferred_element_type=jnp.float32)
    h = jnp.maximum(h + b1_ref[...], 0.0).astype(jnp.bfloat16)
    logit = jax.lax.dot_general(w2_ref[...], h,
                                (((1,), (1,)), ((), ())),
                                preferred_element_type=jnp.float32)
    logit = logit + b2_ref[0, 0]
    out_ref[...] = 5.5 * jax.nn.sigmoid(-logit)


@jax.jit
def _mlp(H, w1, b1, w2, b2):
    grid = (_B // _BLK,)
    return pl.pallas_call(
        _mlp_body,
        grid=grid,
        in_specs=[
            pl.BlockSpec((_BLK, 2 * _D), lambda i: (i, 0)),
            pl.BlockSpec((2 * _D, _H), lambda i: (0, 0)),
            pl.BlockSpec((1, _H), lambda i: (0, 0)),
            pl.BlockSpec((1, _H), lambda i: (0, 0)),
            pl.BlockSpec((1, 1), lambda i: (0, 0)),
        ],
        out_specs=pl.BlockSpec((1, _BLK), lambda i: (0, i)),
        out_shape=jax.ShapeDtypeStruct((1, _B), jnp.float32),
    )(H, w1, b1, w2, b2)


def kernel(x, user_emb, movie_emb, W1, b1, W2, b2):
    uidx = x[:, 0].reshape(_B // _CH, _CH)
    vidx = x[:, 1].reshape(_B // _CH, _CH)
    H = _gather(uidx, vidx, user_emb, movie_emb)
    out = _mlp(H, W1.astype(jnp.bfloat16), b1.reshape(1, _H),
               W2.reshape(1, _H).astype(jnp.bfloat16), b2.reshape(1, 1))
    return out.reshape(_B)


# speedup vs baseline: 1.0677x; 1.0133x over previous
import functools

import jax
import jax.numpy as jnp
from jax import lax
from jax.experimental import pallas as pl
from jax.experimental.pallas import tpu as pltpu
from jax.experimental.pallas import tpu_sc as plsc

_B = 16384
_D = 128
_H = 1024
_NC = 2
_NS = 16
_NW = _NC * _NS
_CH = 128

_NCHUNK = 2
_ROWS = _B // _NCHUNK
_BPW = _ROWS // _NW
_NCH = _BPW // _CH

_BLK = 2048


def _gather_body(uidx_hbm, vidx_hbm, user_hbm, movie_hbm, h_out,
                 idx_v, rows_v, sem):
    wid = lax.axis_index("s") * _NC + lax.axis_index("c")
    base = wid * _BPW
    row0 = wid * _NCH
    for col, idx_hbm, tbl in ((0, uidx_hbm, user_hbm),
                              (_D, vidx_hbm, movie_hbm)):
        pltpu.sync_copy(idx_hbm.at[pl.ds(row0, _NCH)], idx_v)
        copies = [
            pltpu.async_copy(tbl.at[idx_v.at[c]],
                             rows_v.at[pl.ds(c * _CH, _CH)], sem)
            for c in range(_NCH)
        ]
        for cp in copies:
            cp.wait()
        pltpu.sync_copy(rows_v, h_out.at[pl.ds(base, _BPW), pl.ds(col, _D)])


@jax.jit
def _gather(uidx, vidx, user_emb, movie_emb):
    mesh = plsc.VectorSubcoreMesh(core_axis_name="c", subcore_axis_name="s",
                                  num_cores=_NC, num_subcores=_NS)
    return pl.kernel(
        _gather_body,
        out_type=jax.ShapeDtypeStruct((_ROWS, 2 * _D), jnp.float32),
        mesh=mesh,
        scratch_types=[
            pltpu.VMEM((_NCH, _CH), jnp.int32),
            pltpu.VMEM((_BPW, _D), jnp.float32),
            pltpu.SemaphoreType.DMA,
        ],
    )(uidx, vidx, user_emb, movie_emb)


def _mlp_body(h_ref, w1_ref, b1_ref, w2_ref, b2_ref, out_ref):
    hin = jnp.maximum(h_ref[...].astype(jnp.bfloat16), 0)
    h = jnp.dot(hin, w1_ref[...], preferred_element_type=jnp.float32)
    h = jnp.maximum(h + b1_ref[...], 0.0).astype(jnp.bfloat16)
    logit = jax.lax.dot_general(w2_ref[...], h,
                                (((1,), (1,)), ((), ())),
                                preferred_element_type=jnp.float32)
    logit = logit + b2_ref[0, 0]
    out_ref[...] = 5.5 * jax.nn.sigmoid(-logit)


@jax.jit
def _mlp(H, w1, b1, w2, b2):
    grid = (_ROWS // _BLK,)
    return pl.pallas_call(
        _mlp_body,
        grid=grid,
        in_specs=[
            pl.BlockSpec((_BLK, 2 * _D), lambda i: (i, 0)),
            pl.BlockSpec((2 * _D, _H), lambda i: (0, 0)),
            pl.BlockSpec((1, _H), lambda i: (0, 0)),
            pl.BlockSpec((1, _H), lambda i: (0, 0)),
            pl.BlockSpec((1, 1), lambda i: (0, 0)),
        ],
        out_specs=pl.BlockSpec((1, _BLK), lambda i: (0, i)),
        out_shape=jax.ShapeDtypeStruct((1, _ROWS), jnp.float32),
    )(H, w1, b1, w2, b2)


def kernel(x, user_emb, movie_emb, W1, b1, W2, b2):
    uidx = x[:, 0].reshape(_B // _CH, _CH)
    vidx = x[:, 1].reshape(_B // _CH, _CH)
    w1 = W1.astype(jnp.bfloat16)
    b1r = b1.reshape(1, _H)
    w2 = W2.reshape(1, _H).astype(jnp.bfloat16)
    b2r = b2.reshape(1, 1)
    rpc = _ROWS // _CH
    outs = []
    for c in range(_NCHUNK):
        Hc = _gather(uidx[c * rpc:(c + 1) * rpc],
                     vidx[c * rpc:(c + 1) * rpc], user_emb, movie_emb)
        outs.append(_mlp(Hc, w1, b1r, w2, b2r))
    return jnp.concatenate(outs, axis=1).reshape(_B)
